# Initial kernel scaffold; baseline (speedup 1.0000x reference)
import jax
import jax.numpy as jnp
from jax.experimental import pallas as pl


def kernel(sents_batch, emb_weight, lin_weight, lin_bias):
    raise NotImplementedError("write your pallas kernel here")


# speedup vs baseline: 13.1796x; 13.1796x over previous
import functools

import jax
import jax.numpy as jnp
from jax import lax
from jax.experimental import pallas as pl
from jax.experimental.pallas import tpu as pltpu
from jax.experimental.pallas import tpu_sc as plsc

_LANES = 16
_NC = 2
_NS = 16
_NW = _NC * _NS


def _bag_sums_sc(flat_idx, emb_weight, batch, sl):
    emb = emb_weight.shape[1]
    assert emb == 2 * _LANES
    rows_per_w = batch // _NW
    bags_per_chunk = 2
    chunk = bags_per_chunk * sl
    n_chunks = rows_per_w // bags_per_chunk

    mesh = plsc.VectorSubcoreMesh(
        core_axis_name="c", subcore_axis_name="s",
        num_cores=_NC, num_subcores=_NS)

    @functools.partial(
        pl.kernel,
        mesh=mesh,
        compiler_params=pltpu.CompilerParams(use_tc_tiling_on_sc=False),
        out_type=jax.ShapeDtypeStruct((batch, emb), jnp.float32),
        scratch_types=[
            pltpu.VMEM((chunk,), jnp.int32),
            pltpu.VMEM((chunk,), jnp.int32),
            pltpu.VMEM((chunk, emb), jnp.float32),
            pltpu.VMEM((chunk, emb), jnp.float32),
            pltpu.VMEM((rows_per_w, emb), jnp.float32),
            pltpu.SemaphoreType.DMA,
            pltpu.SemaphoreType.DMA,
        ],
    )
    def bag_sum_kernel(idx_hbm, table_hbm, out_hbm,
                       idx0, idx1, buf0, buf1, acc, gsem0, gsem1):
        cid = lax.axis_index("c")
        sid = lax.axis_index("s")
        wid = sid * _NC + cid
        base = wid * rows_per_w * sl

        idx_bufs = (idx0, idx1)
        bufs = (buf0, buf1)
        sems = (gsem0, gsem1)

        def fill(c, b):
            pltpu.sync_copy(idx_hbm.at[pl.ds(base + c * chunk, chunk)],
                            idx_bufs[b])
            pltpu.make_async_copy(table_hbm.at[idx_bufs[b]], bufs[b],
                                  sems[b]).start()

        fill(0, 0)
        fill(1, 1)

        def chunk_body(i, _):
            c = i * 2
            for b in range(2):
                cc = c + b
                pltpu.make_async_copy(table_hbm.at[idx_bufs[b]], bufs[b],
                                      sems[b]).wait()
                for r in range(bags_per_chunk):
                    def tok_body(j, carry, _r=r, _b=b):
                        a0, a1 = carry
                        row = _r * sl + j
                        a0 = a0 + bufs[_b][row, pl.ds(0, _LANES)]
                        a1 = a1 + bufs[_b][row, pl.ds(_LANES, _LANES)]
                        return a0, a1
                    a0, a1 = lax.fori_loop(
                        0, sl, tok_body,
                        (jnp.zeros((_LANES,), jnp.float32),
                         jnp.zeros((_LANES,), jnp.float32)))
                    arow = cc * bags_per_chunk + r
                    acc[arow, pl.ds(0, _LANES)] = a0
                    acc[arow, pl.ds(_LANES, _LANES)] = a1
                nxt = cc + 2

                @pl.when(nxt < n_chunks)
                def _(_b=b, _nxt=nxt):
                    fill(_nxt, _b)
            return 0

        lax.fori_loop(0, n_chunks // 2, chunk_body, 0)
        pltpu.sync_copy(acc, out_hbm.at[pl.ds(wid * rows_per_w, rows_per_w)])

    return bag_sum_kernel(flat_idx, emb_weight)


def _linear_tc(bag_sums, wt_pad, bias_pad, scale):
    batch = bag_sums.shape[0]
    npad = wt_pad.shape[1]

    def mm(x_ref, w_ref, b_ref, o_ref):
        x = x_ref[...] * scale
        o_ref[...] = (
            jnp.dot(x, w_ref[...], preferred_element_type=jnp.float32)
            + b_ref[...])

    return pl.pallas_call(
        mm,
        out_shape=jax.ShapeDtypeStruct((batch, npad), jnp.float32),
    )(bag_sums, wt_pad, bias_pad)


def kernel(sents_batch, emb_weight, lin_weight, lin_bias):
    batch, s, l = sents_batch.shape
    sl = s * l
    flat_idx = sents_batch.reshape(batch * sl)
    sums = _bag_sums_sc(flat_idx, emb_weight, batch, sl)

    classes = lin_weight.shape[0]
    npad = 128
    wt = lin_weight.T
    wt_pad = jnp.zeros((wt.shape[0], npad), wt.dtype).at[:, :classes].set(wt)
    bias_pad = (jnp.zeros((1, npad), lin_bias.dtype)
                .at[0, :classes].set(lin_bias))
    out = _linear_tc(sums, wt_pad, bias_pad, 1.0 / sl)
    return out[:, :classes]


# speedup vs baseline: 15.8618x; 1.2035x over previous
import functools

import jax
import jax.numpy as jnp
from jax import lax
from jax.experimental import pallas as pl
from jax.experimental.pallas import tpu as pltpu
from jax.experimental.pallas import tpu_sc as plsc

_LANES = 16
_NC = 2
_NS = 16
_NW = _NC * _NS


def _bag_sums_sc(flat_idx, emb_weight, batch, sl):
    emb = emb_weight.shape[1]
    assert emb == 2 * _LANES
    assert sl % 2 == 0
    half = sl // 2
    rows_per_w = batch // _NW
    n_chunks = rows_per_w

    mesh = plsc.VectorSubcoreMesh(
        core_axis_name="c", subcore_axis_name="s",
        num_cores=_NC, num_subcores=_NS)

    @functools.partial(
        pl.kernel,
        mesh=mesh,
        compiler_params=pltpu.CompilerParams(use_tc_tiling_on_sc=False),
        out_type=jax.ShapeDtypeStruct((batch, emb), jnp.float32),
        scratch_types=[
            pltpu.VMEM((rows_per_w * sl,), jnp.int32),
            pltpu.VMEM((sl, emb), jnp.float32),
            pltpu.VMEM((sl, emb), jnp.float32),
            pltpu.VMEM((rows_per_w, emb), jnp.float32),
            pltpu.SemaphoreType.DMA,
            pltpu.SemaphoreType.DMA,
        ],
    )
    def bag_sum_kernel(idx_hbm, table_hbm, out_hbm,
                       idx_all, buf0, buf1, acc, gsem0, gsem1):
        cid = lax.axis_index("c")
        sid = lax.axis_index("s")
        wid = sid * _NC + cid
        base = wid * rows_per_w * sl

        bufs = (buf0, buf1)
        sems = (gsem0, gsem1)

        pltpu.sync_copy(idx_hbm.at[pl.ds(base, rows_per_w * sl)], idx_all)

        def fill(c, b):
            pltpu.make_async_copy(
                table_hbm.at[idx_all.at[pl.ds(c * sl, sl)]], bufs[b],
                sems[b]).start()

        fill(0, 0)
        fill(1, 1)

        def chunk_body(cc, _):
            b = lax.rem(cc, 2)

            def reduce_from(_b):
                buf = bufs[_b]
                pltpu.make_async_copy(
                    table_hbm.at[idx_all.at[pl.ds(cc * sl, sl)]], buf,
                    sems[_b]).wait()

                def tok_body(j, carry):
                    a0, a1, b0, b1 = carry
                    a0 = a0 + buf[j, pl.ds(0, _LANES)]
                    a1 = a1 + buf[j, pl.ds(_LANES, _LANES)]
                    b0 = b0 + buf[half + j, pl.ds(0, _LANES)]
                    b1 = b1 + buf[half + j, pl.ds(_LANES, _LANES)]
                    return a0, a1, b0, b1

                z = jnp.zeros((_LANES,), jnp.float32)
                a0, a1, b0, b1 = lax.fori_loop(
                    0, half, tok_body, (z, z, z, z), unroll=4)
                acc[cc, pl.ds(0, _LANES)] = a0 + b0
                acc[cc, pl.ds(_LANES, _LANES)] = a1 + b1

                nxt = cc + 2

                @pl.when(nxt < n_chunks)
                def _():
                    fill(nxt, _b)

            @pl.when(b == 0)
            def _():
                reduce_from(0)

            @pl.when(b == 1)
            def _():
                reduce_from(1)

            return 0

        lax.fori_loop(0, n_chunks, chunk_body, 0)
        pltpu.sync_copy(acc, out_hbm.at[pl.ds(wid * rows_per_w, rows_per_w)])

    return bag_sum_kernel(flat_idx, emb_weight)


def _linear_tc(bag_sums, wt_pad, bias_pad, scale):
    batch = bag_sums.shape[0]
    npad = wt_pad.shape[1]

    def mm(x_ref, w_ref, b_ref, o_ref):
        x = x_ref[...] * scale
        o_ref[...] = (
            jnp.dot(x, w_ref[...], preferred_element_type=jnp.float32)
            + b_ref[...])

    return pl.pallas_call(
        mm,
        out_shape=jax.ShapeDtypeStruct((batch, npad), jnp.float32),
    )(bag_sums, wt_pad, bias_pad)


def kernel(sents_batch, emb_weight, lin_weight, lin_bias):
    batch, s, l = sents_batch.shape
    sl = s * l
    flat_idx = sents_batch.reshape(batch * sl)
    sums = _bag_sums_sc(flat_idx, emb_weight, batch, sl)

    classes = lin_weight.shape[0]
    npad = 128
    wt = lin_weight.T
    wt_pad = jnp.zeros((wt.shape[0], npad), wt.dtype).at[:, :classes].set(wt)
    bias_pad = (jnp.zeros((1, npad), lin_bias.dtype)
                .at[0, :classes].set(lin_bias))
    out = _linear_tc(sums, wt_pad, bias_pad, 1.0 / sl)
    return out[:, :classes]
